# baseline (speedup 1.0000x reference)
import jax
import jax.numpy as jnp
from jax import lax
from jax.experimental import pallas as pl
from jax.experimental.pallas import tpu as pltpu
from jax.experimental.pallas import tpu_sc as plsc

B, C, H, W = 4, 32, 128, 128
HW = H * W
K = 9
M = 9 * 64 * 64
J = K * M
CH = 4608
NCHUNK = M // CH
NTOT = K * NCHUNK
LANES = 16
TPC = 4
NW = 32
WPB = NW // B
UNROLL = 8

FTOT = B * J
FPS = FTOT // NW
FCH = 2304
FNC = FPS // FCH
SPB = J // FPS


def _body(xf, yf, xif, out, ifl, t0, t1, t2, t3, ibuf, obuf, fybuf, fxbuf,
          fobuf, stab, sl0, sl1, ss0, ss1):
    tables = (t0, t1, t2, t3)
    semld = (sl0, sl1)
    semst = (ss0, ss1)
    cid = lax.axis_index("c")
    sid = lax.axis_index("s")
    b = cid * 2 + sid // 8
    cbase = (sid % 8) * TPC

    for j in range(TPC):
        pltpu.async_copy(xf.at[b, cbase + j], tables[j], stab)

    r = cid * 16 + sid
    fb = r // SPB
    foff = (r % SPB) * FPS

    def fld(n, q):
        pltpu.async_copy(yf.at[r, pl.ds(n * FCH, FCH)], fybuf.at[q],
                         semld[q])
        pltpu.async_copy(xif.at[r, pl.ds(n * FCH, FCH)], fxbuf.at[q],
                         semld[q])

    def fld_wait(n, q):
        pltpu.make_async_copy(
            yf.at[r, pl.ds(n * FCH, FCH)], fybuf.at[q], semld[q]).wait()
        pltpu.make_async_copy(
            xif.at[r, pl.ds(n * FCH, FCH)], fxbuf.at[q], semld[q]).wait()

    def fst(n, q):
        pltpu.async_copy(fobuf.at[q],
                         ifl.at[fb, pl.ds(foff + n * FCH, FCH)], semst[q])

    def fst_wait(n, q):
        pltpu.make_async_copy(
            fobuf.at[q], ifl.at[fb, pl.ds(foff + n * FCH, FCH)],
            semst[q]).wait()

    def fcompute(q):
        @plsc.parallel_loop(0, FCH, LANES, unroll=UNROLL)
        def _(off):
            yv = fybuf[q, pl.ds(off, LANES)]
            xv = fxbuf[q, pl.ds(off, LANES)]
            fobuf[q, pl.ds(off, LANES)] = xv + yv * W

    fld(0, 0)
    fld(1, 1)
    fld_wait(0, 0)
    fcompute(0)
    fld(2, 0)
    fst(0, 0)
    fld_wait(1, 1)
    fcompute(1)
    fld(3, 1)
    fst(1, 1)

    def fpair(p, carry):
        for q in (0, 1):
            n = 2 * p + q
            fld_wait(n, q)
            fst_wait(n - 2, q)
            fcompute(q)

            @pl.when(n + 2 <= FNC - 1)
            def _():
                fld(n + 2, q)

            fst(n, q)
        return carry

    lax.fori_loop(1, FNC // 2, fpair, 0)

    fst_wait(FNC - 2, 0)
    fst_wait(FNC - 1, 1)

    for j in range(TPC):
        pltpu.make_async_copy(xf.at[b, cbase + j], tables[j], stab).wait()
    plsc.subcore_barrier()

    def ld(n, q):
        pltpu.async_copy(ifl.at[b, pl.ds(n * CH, CH)], ibuf.at[q], semld[q])

    def ld_wait(n, q):
        pltpu.make_async_copy(
            ifl.at[b, pl.ds(n * CH, CH)], ibuf.at[q], semld[q]).wait()

    def out_slice(n):
        k = n // NCHUNK
        t = n % NCHUNK
        return out.at[b, k, pl.ds(cbase, TPC), pl.ds(t * CH, CH)]

    def st(n, q):
        pltpu.async_copy(obuf.at[q], out_slice(n), semst[q])

    def st_wait(n, q):
        pltpu.make_async_copy(obuf.at[q], out_slice(n), semst[q]).wait()

    def compute(q):
        @plsc.parallel_loop(0, CH, LANES, unroll=UNROLL)
        def _(off):
            iv = ibuf[q, pl.ds(off, LANES)]
            for j in range(TPC):
                obuf[q, j, pl.ds(off, LANES)] = plsc.load_gather(
                    tables[j], [iv])

    ld(0, 0)
    ld(1, 1)
    ld_wait(0, 0)
    compute(0)
    ld(2, 0)
    st(0, 0)
    ld_wait(1, 1)
    compute(1)
    ld(3, 1)
    st(1, 1)

    def pair(p, carry):
        for q in (0, 1):
            n = 2 * p + q
            ld_wait(n, q)
            st_wait(n - 2, q)
            compute(q)

            @pl.when(n + 2 <= NTOT - 1)
            def _():
                ld(n + 2, q)

            st(n, q)
        return carry

    lax.fori_loop(1, NTOT // 2, pair, 0)

    st_wait(NTOT - 2, 0)
    st_wait(NTOT - 1, 1)


@jax.jit
def kernel(x, inref_y, inref_x):
    xf = x.reshape(B, C, HW)
    yflat = inref_y.reshape(NW, FPS)
    xiflat = inref_x.reshape(NW, FPS)
    mesh = plsc.VectorSubcoreMesh(core_axis_name="c", subcore_axis_name="s")
    out, _ = pl.kernel(
        _body,
        out_type=(jax.ShapeDtypeStruct((B, K, C, M), jnp.float32),
                  jax.ShapeDtypeStruct((B, J), jnp.int32)),
        mesh=mesh,
        compiler_params=pltpu.CompilerParams(needs_layout_passes=False),
        scratch_types=[
            pltpu.VMEM((HW,), jnp.float32),
            pltpu.VMEM((HW,), jnp.float32),
            pltpu.VMEM((HW,), jnp.float32),
            pltpu.VMEM((HW,), jnp.float32),
            pltpu.VMEM((2, CH), jnp.int32),
            pltpu.VMEM((2, TPC, CH), jnp.float32),
            pltpu.VMEM((2, FCH), jnp.int32),
            pltpu.VMEM((2, FCH), jnp.int32),
            pltpu.VMEM((2, FCH), jnp.int32),
            pltpu.SemaphoreType.DMA,
            pltpu.SemaphoreType.DMA,
            pltpu.SemaphoreType.DMA,
            pltpu.SemaphoreType.DMA,
            pltpu.SemaphoreType.DMA,
        ],
    )(xf, yflat, xiflat)
    return out


# speedup vs baseline: 1.0863x; 1.0863x over previous
import jax
import jax.numpy as jnp
from jax import lax
from jax.experimental import pallas as pl
from jax.experimental.pallas import tpu as pltpu
from jax.experimental.pallas import tpu_sc as plsc

B, C, H, W = 4, 32, 128, 128
HW = H * W
K = 9
M = 9 * 64 * 64
J = K * M
CH = 4608
NCHUNK = M // CH
NTOT = K * NCHUNK
LANES = 16
TPC = 4
NW = 32
WPB = NW // B
UNROLL = 8

FTOT = B * J
FPS = FTOT // NW
FCH = 2304
FNC = FPS // FCH
SPB = J // FPS
J2 = J // 2
CH2 = CH // 2
FCH2 = FCH // 2


def _body(xf, yf, xif, out, ifl, t0, t1, t2, t3, ibuf, obuf, fybuf, fxbuf,
          fobuf, stab, sl0, sl1, ss0, ss1):
    tables = (t0, t1, t2, t3)
    semld = (sl0, sl1)
    semst = (ss0, ss1)
    cid = lax.axis_index("c")
    sid = lax.axis_index("s")
    b = cid * 2 + sid // 8
    cbase = (sid % 8) * TPC

    for j in range(TPC):
        pltpu.async_copy(xf.at[b, cbase + j], tables[j], stab)

    r = cid * 16 + sid
    fb = r // SPB
    foff = (r % SPB) * FPS
    foff2 = (r % SPB) * (FPS // 2)

    def fld(n, q):
        pltpu.async_copy(yf.at[r, pl.ds(n * FCH, FCH)], fybuf.at[q],
                         semld[q])
        pltpu.async_copy(xif.at[r, pl.ds(n * FCH, FCH)], fxbuf.at[q],
                         semld[q])

    def fld_wait(n, q):
        pltpu.make_async_copy(
            yf.at[r, pl.ds(n * FCH, FCH)], fybuf.at[q], semld[q]).wait()
        pltpu.make_async_copy(
            xif.at[r, pl.ds(n * FCH, FCH)], fxbuf.at[q], semld[q]).wait()

    def fst(n, q):
        pltpu.async_copy(fobuf.at[q],
                         ifl.at[fb, pl.ds(foff2 + n * FCH2, FCH2)],
                         semst[q])

    def fst_wait(n, q):
        pltpu.make_async_copy(
            fobuf.at[q], ifl.at[fb, pl.ds(foff2 + n * FCH2, FCH2)],
            semst[q]).wait()

    def fcompute(q):
        @plsc.parallel_loop(0, FCH, 2 * LANES, unroll=UNROLL)
        def _(off):
            y0 = fybuf[q, pl.ds(off, LANES)]
            x0 = fxbuf[q, pl.ds(off, LANES)]
            y1 = fybuf[q, pl.ds(off + LANES, LANES)]
            x1 = fxbuf[q, pl.ds(off + LANES, LANES)]
            f0 = x0 + y0 * W
            f1 = x1 + y1 * W
            fobuf[q, pl.ds(pl.multiple_of(off // 2, LANES), LANES)] = (
                f0 | (f1 << 16))

    fld(0, 0)
    fld(1, 1)
    fld_wait(0, 0)
    fcompute(0)
    fld(2, 0)
    fst(0, 0)
    fld_wait(1, 1)
    fcompute(1)
    fld(3, 1)
    fst(1, 1)

    def fpair(p, carry):
        for q in (0, 1):
            n = 2 * p + q
            fld_wait(n, q)
            fst_wait(n - 2, q)
            fcompute(q)

            @pl.when(n + 2 <= FNC - 1)
            def _():
                fld(n + 2, q)

            fst(n, q)
        return carry

    lax.fori_loop(1, FNC // 2, fpair, 0)

    fst_wait(FNC - 2, 0)
    fst_wait(FNC - 1, 1)

    for j in range(TPC):
        pltpu.make_async_copy(xf.at[b, cbase + j], tables[j], stab).wait()
    plsc.subcore_barrier()

    def ld(n, q):
        pltpu.async_copy(ifl.at[b, pl.ds(n * CH2, CH2)], ibuf.at[q],
                         semld[q])

    def ld_wait(n, q):
        pltpu.make_async_copy(
            ifl.at[b, pl.ds(n * CH2, CH2)], ibuf.at[q], semld[q]).wait()

    def out_slice(n):
        k = n // NCHUNK
        t = n % NCHUNK
        return out.at[b, k, pl.ds(cbase, TPC), pl.ds(t * CH, CH)]

    def st(n, q):
        pltpu.async_copy(obuf.at[q], out_slice(n), semst[q])

    def st_wait(n, q):
        pltpu.make_async_copy(obuf.at[q], out_slice(n), semst[q]).wait()

    def compute(q):
        @plsc.parallel_loop(0, CH, 2 * LANES, unroll=UNROLL)
        def _(off):
            w = ibuf[q, pl.ds(pl.multiple_of(off // 2, LANES), LANES)]
            lo = w & 0xFFFF
            hi = w >> 16
            for j in range(TPC):
                obuf[q, j, pl.ds(off, LANES)] = plsc.load_gather(
                    tables[j], [lo])
                obuf[q, j, pl.ds(off + LANES, LANES)] = plsc.load_gather(
                    tables[j], [hi])

    ld(0, 0)
    ld(1, 1)
    ld_wait(0, 0)
    compute(0)
    ld(2, 0)
    st(0, 0)
    ld_wait(1, 1)
    compute(1)
    ld(3, 1)
    st(1, 1)

    def pair(p, carry):
        for q in (0, 1):
            n = 2 * p + q
            ld_wait(n, q)
            st_wait(n - 2, q)
            compute(q)

            @pl.when(n + 2 <= NTOT - 1)
            def _():
                ld(n + 2, q)

            st(n, q)
        return carry

    lax.fori_loop(1, NTOT // 2, pair, 0)

    st_wait(NTOT - 2, 0)
    st_wait(NTOT - 1, 1)


@jax.jit
def kernel(x, inref_y, inref_x):
    xf = x.reshape(B, C, HW)
    yflat = inref_y.reshape(NW, FPS)
    xiflat = inref_x.reshape(NW, FPS)
    mesh = plsc.VectorSubcoreMesh(core_axis_name="c", subcore_axis_name="s")
    out, _ = pl.kernel(
        _body,
        out_type=(jax.ShapeDtypeStruct((B, K, C, M), jnp.float32),
                  jax.ShapeDtypeStruct((B, J2), jnp.int32)),
        mesh=mesh,
        compiler_params=pltpu.CompilerParams(needs_layout_passes=False),
        scratch_types=[
            pltpu.VMEM((HW,), jnp.float32),
            pltpu.VMEM((HW,), jnp.float32),
            pltpu.VMEM((HW,), jnp.float32),
            pltpu.VMEM((HW,), jnp.float32),
            pltpu.VMEM((2, CH2), jnp.int32),
            pltpu.VMEM((2, TPC, CH), jnp.float32),
            pltpu.VMEM((2, FCH), jnp.int32),
            pltpu.VMEM((2, FCH), jnp.int32),
            pltpu.VMEM((2, FCH2), jnp.int32),
            pltpu.SemaphoreType.DMA,
            pltpu.SemaphoreType.DMA,
            pltpu.SemaphoreType.DMA,
            pltpu.SemaphoreType.DMA,
            pltpu.SemaphoreType.DMA,
        ],
    )(xf, yflat, xiflat)
    return out
